# baseline (speedup 1.0000x reference)
import functools

import jax
import jax.numpy as jnp
from jax import lax
from jax.experimental import pallas as pl
from jax.experimental.pallas import tpu as pltpu
from jax.experimental.pallas import tpu_sc as plsc

VOCAB = 1000000
EMBED = 64
ROWS = 4096
COLS = 200
WPAD = 128

_info = plsc.get_sparse_core_info()
NC = _info.num_cores
NS = _info.num_subcores
NW = NC * NS
RBLK = ROWS // NW
NBUF = 4
NT = 2
NGRP = COLS // NBUF

_mesh = plsc.VectorSubcoreMesh(core_axis_name="c", subcore_axis_name="s")


@functools.partial(
    pl.kernel,
    mesh=_mesh,
    out_type=jax.ShapeDtypeStruct((COLS, EMBED, ROWS), jnp.float32),
    scratch_types=[
        pltpu.VMEM((COLS, RBLK), jnp.int32),
        pltpu.VMEM((NBUF, RBLK, WPAD), jnp.float32),
        pltpu.VMEM((NT, EMBED, RBLK), jnp.float32),
        pltpu.SemaphoreType.DMA((NBUF,)),
        pltpu.SemaphoreType.DMA((NT,)),
    ],
    compiler_params=pltpu.CompilerParams(
        use_tc_tiling_on_sc=True, needs_layout_passes=False
    ),
)
def _emb_lookup(xt_hbm, wp_hbm, out_hbm, idx_v, g_v, t_v, gsems, osems):
    wid = lax.axis_index("s") * NC + lax.axis_index("c")
    r0 = wid * RBLK
    pltpu.sync_copy(xt_hbm.at[:, pl.ds(r0, RBLK)], idx_v)

    def fire_gather(b, c):
        pltpu.async_copy(wp_hbm.at[idx_v.at[c]], g_v.at[b], gsems.at[b])

    def wait_gather(b):
        pltpu.make_async_copy(
            wp_hbm.at[idx_v.at[0]], g_v.at[b], gsems.at[b]
        ).wait()

    def fire_out(bt, c):
        pltpu.async_copy(
            t_v.at[bt], out_hbm.at[c, :, pl.ds(r0, RBLK)], osems.at[bt]
        )

    def wait_out(bt):
        pltpu.make_async_copy(
            t_v.at[bt], out_hbm.at[0, :, pl.ds(r0, RBLK)], osems.at[bt]
        ).wait()

    lanes = lax.iota(jnp.int32, 16)

    def transpose(b, bt):
        g = g_v.at[b]
        t = t_v.at[bt]

        def ebody(eo, carry):
            for ei in range(8):
                e = eo * 8 + ei
                col = jnp.full((16,), 0, jnp.int32) + e
                for lg in range(8):
                    vals = plsc.load_gather(g, [lanes + 16 * lg, col])
                    t[e, pl.ds(lg * 16, 16)] = vals
            return carry

        lax.fori_loop(0, 8, ebody, 0)

    def chunk(c, b, bt, first_t, fire_next):
        wait_gather(b)
        if not first_t:
            wait_out(bt)
        transpose(b, bt)
        fire_out(bt, c)
        if fire_next:
            fire_gather(b, c + NBUF)

    for b in range(NBUF):
        fire_gather(b, b)
    for b in range(NBUF):
        chunk(b, b, b % NT, first_t=(b < NT), fire_next=True)

    def group(g, carry):
        for b in range(NBUF):
            c = g * NBUF + b
            chunk(c, b, b % NT, first_t=False, fire_next=True)
        return carry

    lax.fori_loop(1, NGRP - 1, group, 0)

    for b in range(NBUF):
        c = (NGRP - 1) * NBUF + b
        chunk(c, b, b % NT, first_t=False, fire_next=False)
    for bt in range(NT):
        wait_out(bt)


def kernel(x, w):
    xt = jnp.transpose(x, (1, 0))
    wp = jnp.pad(w, ((0, 0), (0, WPAD - EMBED)))
    out_p = _emb_lookup(xt, wp)
    return jnp.transpose(out_p, (2, 0, 1))


# speedup vs baseline: 1.2032x; 1.2032x over previous
import functools

import jax
import jax.numpy as jnp
from jax import lax
from jax.experimental import pallas as pl
from jax.experimental.pallas import tpu as pltpu
from jax.experimental.pallas import tpu_sc as plsc

VOCAB = 1000000
EMBED = 64
ROWS = 4096
COLS = 200
WPAIR = 128

_info = plsc.get_sparse_core_info()
NC = _info.num_cores
NS = _info.num_subcores
NW = NC * NS
RBLK = ROWS // NW
NBUF = 3
NT = 2
NGRP = COLS // NBUF
NTAIL = COLS - (NGRP - 1) * NBUF - NBUF

_mesh = plsc.VectorSubcoreMesh(core_axis_name="c", subcore_axis_name="s")


@functools.partial(
    pl.kernel,
    mesh=_mesh,
    out_type=jax.ShapeDtypeStruct((COLS, EMBED, ROWS), jnp.float32),
    scratch_types=[
        pltpu.VMEM((COLS, RBLK), jnp.int32),
        pltpu.VMEM((NBUF, RBLK, WPAIR), jnp.float32),
        pltpu.VMEM((NT, EMBED, RBLK), jnp.float32),
        pltpu.VMEM((NBUF, RBLK), jnp.int32),
        pltpu.VMEM((NBUF, RBLK), jnp.int32),
        pltpu.VMEM((16, 16), jnp.float32),
        pltpu.SemaphoreType.DMA((NBUF,)),
        pltpu.SemaphoreType.DMA((NT,)),
    ],
    compiler_params=pltpu.CompilerParams(
        use_tc_tiling_on_sc=True, needs_layout_passes=False
    ),
)
def _emb_lookup(
    xt_hbm, wp_hbm, out_hbm, idx_v, g_v, t_v, u_v, h_v, s_v, gsems, osems
):
    wid = lax.axis_index("s") * NC + lax.axis_index("c")
    r0 = wid * RBLK
    pltpu.sync_copy(xt_hbm.at[:, pl.ds(r0, RBLK)], idx_v)

    lanes = lax.iota(jnp.int32, 16)
    rot = [(lanes + k) & 15 for k in range(16)]
    jrot = [(jnp.int32(j) - lanes) & 15 for j in range(16)]

    def prep(b, c):
        for lg in range(8):
            iv = idx_v[c, pl.ds(16 * lg, 16)]
            u_v[b, pl.ds(16 * lg, 16)] = lax.shift_right_logical(iv, 1)
            h_v[b, pl.ds(16 * lg, 16)] = lax.shift_left(iv & 1, 6)

    def fire_gather(b, c):
        pltpu.async_copy(wp_hbm.at[u_v.at[b]], g_v.at[b], gsems.at[b])

    def wait_gather(b):
        pltpu.make_async_copy(
            wp_hbm.at[u_v.at[0]], g_v.at[b], gsems.at[b]
        ).wait()

    def fire_out(bt, c):
        pltpu.async_copy(
            t_v.at[bt], out_hbm.at[c, :, pl.ds(r0, RBLK)], osems.at[bt]
        )

    def wait_out(bt):
        pltpu.make_async_copy(
            t_v.at[bt], out_hbm.at[0, :, pl.ds(r0, RBLK)], osems.at[bt]
        ).wait()

    def transpose(b, bt):
        g = g_v.at[b]

        def body(i, carry):
            e0 = lax.shift_left(lax.shift_right_logical(i, 3), 4)
            l0 = pl.multiple_of(lax.shift_left(i & 7, 4), 16)
            hv = h_v[b, pl.ds(l0, 16)]
            hve = hv + e0
            row = lanes + l0
            for k in range(16):
                s_v[k, :] = plsc.load_gather(g, [row, hve + rot[k]])
            for j in range(16):
                t_v[bt, e0 + j, pl.ds(l0, 16)] = plsc.load_gather(
                    s_v, [jrot[j], lanes]
                )
            return carry

        lax.fori_loop(0, 32, body, 0)

    def chunk(c, b, bt, first_t, fire_next):
        wait_gather(b)
        if not first_t:
            wait_out(bt)
        transpose(b, bt)
        fire_out(bt, c)
        if fire_next:
            prep(b, c + NBUF)
            fire_gather(b, c + NBUF)

    for b in range(NBUF):
        prep(b, b)
        fire_gather(b, b)
    for b in range(NBUF):
        chunk(b, b, b % NT, first_t=(b < NT), fire_next=True)

    def group(g, carry):
        for b in range(NBUF):
            c = g * NBUF + b
            chunk(c, b, (g * NBUF + b) % NT, first_t=False, fire_next=True)
        return carry

    lax.fori_loop(1, NGRP - 1, group, 0)

    for i in range(NBUF + NTAIL):
        c = (NGRP - 1) * NBUF + i
        chunk(c, c % NBUF, c % NT, first_t=False,
              fire_next=(c + NBUF < COLS))
    for bt in range(NT):
        wait_out(bt)


def kernel(x, w):
    xt = jnp.transpose(x, (1, 0))
    wp = jnp.reshape(w, (VOCAB // 2, WPAIR))
    out_p = _emb_lookup(xt, wp)
    return jnp.transpose(out_p, (2, 0, 1))


# speedup vs baseline: 1.4394x; 1.1963x over previous
import functools

import jax
import jax.numpy as jnp
from jax import lax
from jax.experimental import pallas as pl
from jax.experimental.pallas import tpu as pltpu
from jax.experimental.pallas import tpu_sc as plsc

VOCAB = 1000000
EMBED = 64
ROWS = 4096
COLS = 200

_info = plsc.get_sparse_core_info()
NC = _info.num_cores
NS = _info.num_subcores
NW = NC * NS
ROWS_W = ROWS // NW
CHA = 128
CHB = COLS - CHA
NBUF = 8
T = ROWS_W * 2
NGRP = T // NBUF

_mesh = plsc.VectorSubcoreMesh(core_axis_name="c", subcore_axis_name="s")


@functools.partial(
    pl.kernel,
    mesh=_mesh,
    out_type=jax.ShapeDtypeStruct((ROWS, COLS, EMBED), jnp.float32),
    scratch_types=[
        pltpu.VMEM((ROWS_W, COLS), jnp.int32),
        pltpu.VMEM((NBUF // 2, CHA, EMBED), jnp.float32),
        pltpu.VMEM((NBUF // 2, CHB, EMBED), jnp.float32),
        pltpu.SemaphoreType.DMA((NBUF,)),
        pltpu.SemaphoreType.DMA((NBUF,)),
    ],
    compiler_params=pltpu.CompilerParams(use_tc_tiling_on_sc=False),
)
def _emb_lookup(x_hbm, w_hbm, out_hbm, idx_v, rows_a, rows_b, gsems, osems):
    wid = lax.axis_index("s") * NC + lax.axis_index("c")
    r0 = wid * ROWS_W
    pltpu.sync_copy(x_hbm.at[pl.ds(r0, ROWS_W)], idx_v)

    def buf(b):
        if b % 2 == 0:
            return rows_a.at[b // 2], 0, CHA
        return rows_b.at[b // 2], CHA, CHB

    def fire_gather(b, t):
        row = t // 2
        dst, col, ch = buf(b)
        pltpu.async_copy(
            w_hbm.at[idx_v.at[row, pl.ds(col, ch)]], dst, gsems.at[b]
        )

    def wait_gather(b):
        dst, col, ch = buf(b)
        pltpu.make_async_copy(
            w_hbm.at[idx_v.at[0, pl.ds(col, ch)]], dst, gsems.at[b]
        ).wait()

    def fire_out(b, t):
        row = t // 2
        src, col, ch = buf(b)
        pltpu.async_copy(
            src, out_hbm.at[r0 + row, pl.ds(col, ch)], osems.at[b]
        )

    def wait_out(b):
        src, col, ch = buf(b)
        pltpu.make_async_copy(
            src, out_hbm.at[r0, pl.ds(col, ch)], osems.at[b]
        ).wait()

    for b in range(NBUF):
        fire_gather(b, b)

    def group(g, carry):
        for b in range(NBUF):
            wait_gather(b)
            fire_out(b, g * NBUF + b)
        for b in range(NBUF):
            wait_out(b)
            fire_gather(b, (g + 1) * NBUF + b)
        return carry

    lax.fori_loop(0, NGRP - 1, group, 0)

    gl = NGRP - 1
    for b in range(NBUF):
        wait_gather(b)
        fire_out(b, gl * NBUF + b)
    for b in range(NBUF):
        wait_out(b)


def kernel(x, w):
    return _emb_lookup(x, w)
